# baseline (speedup 1.0000x reference)
import dataclasses

import jax
import jax.numpy as jnp
from jax import lax
from jax.experimental import pallas as pl
from jax.experimental.pallas import tpu as pltpu
from jax.experimental.pallas import tpu_sc as plsc

_B = 4096
_V = 100000
_NTOP = 1024
_VT = 1000
_NV = _V // _VT

_N_SUBCORES = 32
_CHUNK = _B // _N_SUBCORES
_LANES = 16


def _sc_params():
    cp = pltpu.CompilerParams()
    if "needs_layout_passes" in pltpu.CompilerParams.__dataclass_fields__:
        cp = dataclasses.replace(cp, needs_layout_passes=False)
    return cp


def _sc_mesh():
    return plsc.VectorSubcoreMesh(core_axis_name="c", subcore_axis_name="s")


def _isin_mask(tokens2d, ids):
    @pl.kernel(out_type=jax.ShapeDtypeStruct((1, _B), jnp.float32),
               mesh=_sc_mesh(),
               scratch_types=[pltpu.VMEM((_V,), jnp.int32),
                              pltpu.VMEM((_NTOP,), jnp.int32),
                              pltpu.VMEM((_CHUNK,), jnp.int32),
                              pltpu.VMEM((_CHUNK,), jnp.float32),
                              pltpu.SemaphoreType.DMA],
               compiler_params=_sc_params())
    def isin_kernel(tokens_hbm, ids_hbm, out_hbm, table, ids_v, toks_v,
                    flags_v, sem):
        sub = lax.axis_index("c") * 16 + lax.axis_index("s")
        base = sub * _CHUNK

        @pl.loop(0, _V, step=_LANES)
        def _(i):
            table[pl.ds(i, _LANES)] = jnp.zeros((_LANES,), jnp.int32)

        pltpu.async_copy(ids_hbm, ids_v, sem).wait()

        @pl.loop(0, _NTOP, step=_LANES)
        def _(i):
            plsc.store_scatter(table, [ids_v[pl.ds(i, _LANES)]],
                               jnp.ones((_LANES,), jnp.int32))

        pltpu.async_copy(tokens_hbm.at[0, pl.ds(base, _CHUNK)], toks_v,
                         sem).wait()

        @pl.loop(0, _CHUNK, step=_LANES)
        def _(i):
            fl = plsc.load_gather(table, [toks_v[pl.ds(i, _LANES)]])
            flags_v[pl.ds(i, _LANES)] = fl.astype(jnp.float32)

        pltpu.async_copy(flags_v, out_hbm.at[0, pl.ds(base, _CHUNK)],
                         sem).wait()

    return isin_kernel(tokens2d, ids)


def _platt_finalize(mask, m, s, xt, gw, gb, tw, tb):
    @pl.kernel(out_type=jax.ShapeDtypeStruct((1, _B), jnp.float32),
               mesh=_sc_mesh(),
               scratch_types=[pltpu.VMEM((_CHUNK,), jnp.float32),
                              pltpu.VMEM((_CHUNK,), jnp.float32),
                              pltpu.VMEM((_CHUNK,), jnp.float32),
                              pltpu.VMEM((_CHUNK,), jnp.float32),
                              pltpu.VMEM((_LANES,), jnp.float32),
                              pltpu.VMEM((_LANES,), jnp.float32),
                              pltpu.VMEM((_LANES,), jnp.float32),
                              pltpu.VMEM((_LANES,), jnp.float32),
                              pltpu.VMEM((_CHUNK,), jnp.float32),
                              pltpu.SemaphoreType.DMA],
               compiler_params=_sc_params())
    def fin_kernel(mask_hbm, m_hbm, s_hbm, xt_hbm, gw_hbm, gb_hbm, tw_hbm,
                   tb_hbm, out_hbm, mask_v, m_v, s_v, xt_v, gw_v, gb_v,
                   tw_v, tb_v, out_v, sem):
        sub = lax.axis_index("c") * 16 + lax.axis_index("s")
        base = sub * _CHUNK
        cols = (0, pl.ds(base, _CHUNK))
        pltpu.async_copy(mask_hbm.at[*cols], mask_v, sem).wait()
        pltpu.async_copy(m_hbm.at[*cols], m_v, sem).wait()
        pltpu.async_copy(s_hbm.at[*cols], s_v, sem).wait()
        pltpu.async_copy(xt_hbm.at[*cols], xt_v, sem).wait()
        pltpu.async_copy(gw_hbm, gw_v, sem).wait()
        pltpu.async_copy(gb_hbm, gb_v, sem).wait()
        pltpu.async_copy(tw_hbm, tw_v, sem).wait()
        pltpu.async_copy(tb_hbm, tb_v, sem).wait()

        @pl.loop(0, _CHUNK, step=_LANES)
        def _(i):
            sl = pl.ds(i, _LANES)
            hit = mask_v[sl] > 0.5
            g = jnp.exp(xt_v[sl] - m_v[sl]) / s_v[sl]
            w = jnp.where(hit, tw_v[...], gw_v[...])
            b = jnp.where(hit, tb_v[...], gb_v[...])
            z = g * w + b
            out_v[sl] = 1.0 / (1.0 + jnp.exp(-z))

        pltpu.async_copy(out_v, out_hbm.at[*cols], sem).wait()

    return fin_kernel(mask, m, s, xt, gw, gb, tw, tb)


def _col_kernel(tokens_ref, x_ref, m_ref, s_ref, xt_ref):
    j = pl.program_id(0)

    @pl.when(j == 0)
    def _():
        m_ref[...] = jnp.full((1, _B), -jnp.inf, jnp.float32)
        s_ref[...] = jnp.zeros((1, _B), jnp.float32)
        xt_ref[...] = jnp.zeros((1, _B), jnp.float32)

    tile = x_ref[...]
    tloc = tokens_ref[...] - j * _VT

    loc = jax.lax.broadcasted_iota(jnp.int32, (_VT, _B), 0)
    xt_ref[...] += jnp.sum(jnp.where(loc == tloc, tile, 0.0),
                           axis=0, keepdims=True)

    m_old = m_ref[...]
    m_new = jnp.maximum(m_old, jnp.max(tile, axis=0, keepdims=True))
    s_ref[...] = (s_ref[...] * jnp.exp(m_old - m_new)
                  + jnp.sum(jnp.exp(tile - m_new), axis=0, keepdims=True))
    m_ref[...] = m_new


def kernel(x, tokens, top_token_ids, gen_w, gen_b, top_w, top_b):
    xt_view = x.T
    tokens2d = tokens.astype(jnp.int32).reshape(1, _B)
    ids = top_token_ids.astype(jnp.int32)

    mask = _isin_mask(tokens2d, ids)

    m, s, xt = pl.pallas_call(
        _col_kernel,
        grid=(_NV,),
        in_specs=[
            pl.BlockSpec((1, _B), lambda j: (0, 0)),
            pl.BlockSpec((_VT, _B), lambda j: (j, 0)),
        ],
        out_specs=[
            pl.BlockSpec((1, _B), lambda j: (0, 0)),
            pl.BlockSpec((1, _B), lambda j: (0, 0)),
            pl.BlockSpec((1, _B), lambda j: (0, 0)),
        ],
        out_shape=[jax.ShapeDtypeStruct((1, _B), jnp.float32)] * 3,
        compiler_params=pltpu.CompilerParams(
            dimension_semantics=("arbitrary",)),
    )(tokens2d, xt_view)

    bc16 = lambda a: jnp.full((_LANES,), a.reshape(()), jnp.float32)
    out = _platt_finalize(mask, m, s, xt, bc16(gen_w), bc16(gen_b),
                          bc16(top_w), bc16(top_b))
    return out.reshape(_B)
